# Initial kernel scaffold; baseline (speedup 1.0000x reference)
import jax
import jax.numpy as jnp
from jax.experimental import pallas as pl


def kernel(x, adj, tar_ei, boolen, W_xslin, b_xslin):
    raise NotImplementedError("write your pallas kernel here")


# speedup vs baseline: 4.7121x; 4.7121x over previous
import functools

import jax
import jax.numpy as jnp
from jax import lax
from jax.experimental import pallas as pl
from jax.experimental.pallas import tpu as pltpu
from jax.experimental.pallas import tpu_sc as plsc

N_NODES = 10000
D_FEAT = 128
N_EDGES = 320000
N_TAR = 4096

NC = 2
NS = 16
LANES = 16

WORDS = 1280
ROWS_PER_PASS = 625
N_PASS = 8
BLOCK_WORDS = ROWS_PER_PASS * WORDS
SENT_PAD = 128
SPMEM_WORDS = BLOCK_WORDS + SENT_PAD
ZSTRIPE = SPMEM_WORDS // NS
FSTRIPE = BLOCK_WORDS // NS
E_PER_TEC = N_EDGES // NS
CHUNK = 80
N_CHUNK = E_PER_TEC // CHUNK

G_ROWS = 2 * N_TAR
G_PER_TEC = G_ROWS // (NC * NS)
GCHUNK = 64
NGCH = G_PER_TEC // GCHUNK

K_PAD = WORDS * 8


FCHUNK = FSTRIPE // 2


def _build_body(src_hbm, dst_hbm, c_hbm, src_v, dst_v, idx_v, val_v, zbuf,
                bounce, spmem):
    cid = lax.axis_index("c")
    sid = lax.axis_index("s")

    ebase = sid * E_PER_TEC
    pltpu.sync_copy(src_hbm.at[pl.ds(ebase, E_PER_TEC)], src_v)
    pltpu.sync_copy(dst_hbm.at[pl.ds(ebase, E_PER_TEC)], dst_v)

    def zinit(i, _):
        zbuf[pl.ds(i * LANES, LANES)] = jnp.zeros((LANES,), jnp.int32)
        return 0
    lax.fori_loop(0, zbuf.shape[0] // LANES, zinit, 0)

    for p in range(N_PASS):
        zoff = sid * ZSTRIPE
        nzfull = ZSTRIPE // 8192
        for i in range(nzfull):
            pltpu.sync_copy(zbuf, spmem.at[pl.ds(zoff + i * 8192, 8192)])
        pltpu.sync_copy(zbuf.at[pl.ds(0, ZSTRIPE - nzfull * 8192)],
                        spmem.at[pl.ds(zoff + nzfull * 8192,
                                       ZSTRIPE - nzfull * 8192)])
        plsc.subcore_barrier()

        row_lo = cid * (ROWS_PER_PASS * N_PASS) + p * ROWS_PER_PASS
        row_hi = row_lo + ROWS_PER_PASS
        sent = BLOCK_WORDS + sid * 8

        def chunk_body(j, _):
            base = j * CHUNK
            for v in range(CHUNK // LANES):
                r = src_v[pl.ds(base + v * LANES, LANES)]
                c = dst_v[pl.ds(base + v * LANES, LANES)]
                inr = (r >= row_lo) & (r < row_hi)
                addr = (r - row_lo) * WORDS + lax.shift_right_logical(c, 3)
                addr = jnp.where(inr, addr, sent)
                shift = lax.shift_left(jnp.bitwise_and(c, 7), 2)
                val = jnp.where(inr, lax.shift_left(1, shift), 0)
                idx_v[pl.ds(v * LANES, LANES)] = addr
                val_v[pl.ds(v * LANES, LANES)] = val
            pltpu.sync_copy(val_v, spmem.at[idx_v], add=True)
            return 0
        lax.fori_loop(0, N_CHUNK, chunk_body, 0)
        plsc.subcore_barrier()

        foff = sid * FSTRIPE
        for f in range(FSTRIPE // FCHUNK):
            pltpu.sync_copy(spmem.at[pl.ds(foff + f * FCHUNK, FCHUNK)],
                            bounce)
            pltpu.sync_copy(
                bounce,
                c_hbm.at[pl.ds(row_lo * WORDS + foff + f * FCHUNK, FCHUNK)])
        plsc.subcore_barrier()


def _gather_body(c_hbm, x_hbm, tix_hbm, g_hbm, xg_hbm, idx_v, rows_v, xrows_v,
                 sem):
    cid = lax.axis_index("c")
    sid = lax.axis_index("s")
    wid = cid * NS + sid
    base = wid * G_PER_TEC
    for ch in range(NGCH):
        off = base + ch * GCHUNK
        pltpu.sync_copy(tix_hbm.at[pl.ds(off, GCHUNK)], idx_v)
        pltpu.async_copy(c_hbm.at[idx_v], rows_v, sem).wait()
        pltpu.sync_copy(rows_v, g_hbm.at[pl.ds(off, GCHUNK)])
        pltpu.async_copy(x_hbm.at[idx_v], xrows_v, sem).wait()
        pltpu.sync_copy(xrows_v, xg_hbm.at[pl.ds(off, GCHUNK)])


def _s_body(x_ref, w2_ref, out_ref):
    out_ref[0, 0, :] = jnp.sum(x_ref[...] * w2_ref[...], axis=1)


def _dense_body(gi_ref, gj_ref, xi_ref, xj_ref, sr_ref, w1_ref, b_ref,
                out_ref):
    gi = gi_ref[...]
    gj = gj_ref[...]
    mask_const = jnp.int32(0x11111111)
    zi = gi | lax.shift_right_logical(gi, 1)
    zi = (zi | lax.shift_right_logical(zi, 2)) & mask_const
    zj = gj | lax.shift_right_logical(gj, 1)
    zj = (zj | lax.shift_right_logical(zj, 2)) & mask_const
    m = zi & zj
    acc = jnp.zeros(gi.shape, jnp.float32)
    for t in range(8):
        bit = lax.shift_right_logical(m, 4 * t) & 1
        acc = acc + bit.astype(jnp.float32) * sr_ref[t, :][None, :]
    cn_term = jnp.sum(acc, axis=1)
    xij = jnp.sum(xi_ref[...] * xj_ref[...] * w1_ref[...], axis=1)
    out_ref[0, 0, :] = cn_term + xij + b_ref[0, 0]


def kernel(x, adj, tar_ei, boolen, W_xslin, b_xslin):
    del boolen
    x = x.astype(jnp.float32)
    adj0 = adj[0].astype(jnp.int32)
    adj1 = adj[1].astype(jnp.int32)
    tcat = jnp.concatenate([tar_ei[0], tar_ei[1]]).astype(jnp.int32)
    w1 = W_xslin[0, :D_FEAT].reshape(1, D_FEAT)
    w2 = W_xslin[0, D_FEAT:].reshape(1, D_FEAT)
    b_arr = b_xslin.reshape(1, 1)

    mesh = plsc.VectorSubcoreMesh(core_axis_name="c", subcore_axis_name="s")

    build = pl.kernel(
        _build_body,
        out_type=jax.ShapeDtypeStruct((N_NODES * WORDS,), jnp.int32),
        mesh=mesh,
        scratch_types=[
            pltpu.VMEM((E_PER_TEC,), jnp.int32),
            pltpu.VMEM((E_PER_TEC,), jnp.int32),
            pltpu.VMEM((CHUNK,), jnp.int32),
            pltpu.VMEM((CHUNK,), jnp.int32),
            pltpu.VMEM((8192,), jnp.int32),
            pltpu.VMEM((FCHUNK,), jnp.int32),
            pltpu.VMEM_SHARED((SPMEM_WORDS,), jnp.int32),
        ],
    )
    c_flat = build(adj0, adj1)
    c_2d = c_flat.reshape(N_NODES, WORDS)

    gather = pl.kernel(
        _gather_body,
        out_type=(
            jax.ShapeDtypeStruct((G_ROWS, WORDS), jnp.int32),
            jax.ShapeDtypeStruct((G_ROWS, D_FEAT), jnp.float32),
        ),
        mesh=mesh,
        scratch_types=[
            pltpu.VMEM((GCHUNK,), jnp.int32),
            pltpu.VMEM((GCHUNK, WORDS), jnp.int32),
            pltpu.VMEM((GCHUNK, D_FEAT), jnp.float32),
            pltpu.SemaphoreType.DMA,
        ],
    )
    g_rows, xg_rows = gather(c_2d, x, tcat)

    x_pad = jnp.pad(x, ((0, K_PAD - N_NODES), (0, 0)))
    s_blocks = pl.pallas_call(
        _s_body,
        grid=(K_PAD // 2048,),
        in_specs=[
            pl.BlockSpec((2048, D_FEAT), lambda i: (i, 0)),
            pl.BlockSpec((1, D_FEAT), lambda i: (0, 0)),
        ],
        out_specs=pl.BlockSpec((1, 1, 2048), lambda i: (i, 0, 0)),
        out_shape=jax.ShapeDtypeStruct((K_PAD // 2048, 1, 2048), jnp.float32),
    )(x_pad, w2)
    s_r = s_blocks.reshape(K_PAD)[: WORDS * 8].reshape(WORDS, 8).T

    gi = g_rows[:N_TAR]
    gj = g_rows[N_TAR:]
    xi = xg_rows[:N_TAR]
    xj = xg_rows[N_TAR:]
    EB = 512
    out_blocks = pl.pallas_call(
        _dense_body,
        grid=(N_TAR // EB,),
        in_specs=[
            pl.BlockSpec((EB, WORDS), lambda i: (i, 0)),
            pl.BlockSpec((EB, WORDS), lambda i: (i, 0)),
            pl.BlockSpec((EB, D_FEAT), lambda i: (i, 0)),
            pl.BlockSpec((EB, D_FEAT), lambda i: (i, 0)),
            pl.BlockSpec((8, WORDS), lambda i: (0, 0)),
            pl.BlockSpec((1, D_FEAT), lambda i: (0, 0)),
            pl.BlockSpec((1, 1), lambda i: (0, 0)),
        ],
        out_specs=pl.BlockSpec((1, 1, EB), lambda i: (i, 0, 0)),
        out_shape=jax.ShapeDtypeStruct((N_TAR // EB, 1, EB), jnp.float32),
    )(gi, gj, xi, xj, s_r, w1, b_arr)
    return out_blocks.reshape(N_TAR, 1)


# speedup vs baseline: 6.6121x; 1.4032x over previous
import functools

import jax
import jax.numpy as jnp
from jax import lax
from jax.experimental import pallas as pl
from jax.experimental.pallas import tpu as pltpu
from jax.experimental.pallas import tpu_sc as plsc

N_NODES = 10000
D_FEAT = 128
N_EDGES = 320000
N_TAR = 4096

NC = 2
NS = 16
LANES = 16

WORDS = 1280
ROWS_PER_PASS = 1250
N_PASS = 4
BLOCK_WORDS = ROWS_PER_PASS * WORDS
SENT_PAD = 128
SPMEM_WORDS = BLOCK_WORDS + SENT_PAD
ZSTRIPE = SPMEM_WORDS // NS
FSTRIPE = BLOCK_WORDS // NS
E_PER_TEC = N_EDGES // NS
MEGA = 2000
N_MEGA = E_PER_TEC // MEGA
MROW = 80
MEGA_ROWS = MEGA // MROW

G_ROWS = 2 * N_TAR
G_PER_TEC = N_TAR // (NC * NS)
GCHUNK = 64
NGCH = G_PER_TEC // GCHUNK

K_PAD = WORDS * 8

ZCHUNK = 4096
FCHUNK = FSTRIPE // 10


def _build_body(src_hbm, dst_hbm, c_hbm, src_v, dst_v, idx_v, val_v, zbuf,
                bounce, spmem, sem):
    cid = lax.axis_index("c")
    sid = lax.axis_index("s")

    def zinit(i, _):
        zbuf[pl.ds(i * LANES, LANES)] = jnp.zeros((LANES,), jnp.int32)
        return 0
    lax.fori_loop(0, ZCHUNK // LANES, zinit, 0)

    for p in range(N_PASS):
        zoff = sid * ZSTRIPE
        nzfull = ZSTRIPE // ZCHUNK
        for i in range(nzfull):
            pltpu.sync_copy(zbuf, spmem.at[pl.ds(zoff + i * ZCHUNK, ZCHUNK)])
        pltpu.sync_copy(zbuf.at[pl.ds(0, ZSTRIPE - nzfull * ZCHUNK)],
                        spmem.at[pl.ds(zoff + nzfull * ZCHUNK,
                                       ZSTRIPE - nzfull * ZCHUNK)])
        plsc.subcore_barrier()

        row_lo = cid * (ROWS_PER_PASS * N_PASS) + p * ROWS_PER_PASS
        row_hi = row_lo + ROWS_PER_PASS
        sent = BLOCK_WORDS + sid * 8

        def mega_body(m, _):
            ebase = sid * E_PER_TEC + m * MEGA
            pltpu.sync_copy(src_hbm.at[pl.ds(ebase, MEGA)], src_v)
            pltpu.sync_copy(dst_hbm.at[pl.ds(ebase, MEGA)], dst_v)

            def row_body(q, _):
                for v in range(MROW // LANES):
                    r = src_v[pl.ds(q * MROW + v * LANES, LANES)]
                    c = dst_v[pl.ds(q * MROW + v * LANES, LANES)]
                    inr = (r >= row_lo) & (r < row_hi)
                    addr = (r - row_lo) * WORDS + lax.shift_right_logical(c, 3)
                    addr = jnp.where(inr, addr, sent)
                    shift = lax.shift_left(jnp.bitwise_and(c, 7), 2)
                    val = jnp.where(inr, lax.shift_left(1, shift), 0)
                    idx_v[q, pl.ds(v * LANES, LANES)] = addr
                    val_v[q, pl.ds(v * LANES, LANES)] = val
                return 0
            lax.fori_loop(0, MEGA_ROWS, row_body, 0)
            copies = [
                pltpu.async_copy(val_v.at[q], spmem.at[idx_v.at[q]], sem,
                                 add=True)
                for q in range(MEGA_ROWS)
            ]
            for cp in copies:
                cp.wait()
            return 0
        lax.fori_loop(0, N_MEGA, mega_body, 0)
        plsc.subcore_barrier()

        foff = sid * FSTRIPE
        for f in range(FSTRIPE // FCHUNK):
            pltpu.sync_copy(spmem.at[pl.ds(foff + f * FCHUNK, FCHUNK)],
                            bounce)
            pltpu.sync_copy(
                bounce,
                c_hbm.at[pl.ds(row_lo * WORDS + foff + f * FCHUNK, FCHUNK)])
        plsc.subcore_barrier()


def _gather_body(c_hbm, x_hbm, t0_hbm, t1_hbm, gi_hbm, gj_hbm, xi_hbm, xj_hbm,
                 idx_v, rows_v, xrows_v, sem):
    cid = lax.axis_index("c")
    sid = lax.axis_index("s")
    wid = cid * NS + sid
    base = wid * G_PER_TEC
    for t_hbm, g_hbm, xg_hbm in ((t0_hbm, gi_hbm, xi_hbm),
                                 (t1_hbm, gj_hbm, xj_hbm)):
        for ch in range(NGCH):
            off = base + ch * GCHUNK
            pltpu.sync_copy(t_hbm.at[pl.ds(off, GCHUNK)], idx_v)
            pltpu.async_copy(c_hbm.at[idx_v], rows_v, sem).wait()
            pltpu.sync_copy(rows_v, g_hbm.at[pl.ds(off, GCHUNK)])
            pltpu.async_copy(x_hbm.at[idx_v], xrows_v, sem).wait()
            pltpu.sync_copy(xrows_v, xg_hbm.at[pl.ds(off, GCHUNK)])


def _s_body(x_ref, w2_ref, out_ref):
    out_ref[0, 0, :] = jnp.sum(x_ref[...] * w2_ref[...], axis=1)


def _dense_body(gi_ref, gj_ref, xi_ref, xj_ref, sr_ref, w1_ref, b_ref,
                out_ref):
    gi = gi_ref[...]
    gj = gj_ref[...]
    mask_const = jnp.int32(0x11111111)
    zi = gi | lax.shift_right_logical(gi, 1)
    zi = (zi | lax.shift_right_logical(zi, 2)) & mask_const
    zj = gj | lax.shift_right_logical(gj, 1)
    zj = (zj | lax.shift_right_logical(zj, 2)) & mask_const
    m = zi & zj
    acc = jnp.zeros(gi.shape, jnp.float32)
    for t in range(8):
        bit = lax.shift_right_logical(m, 4 * t) & 1
        acc = acc + bit.astype(jnp.float32) * sr_ref[t, :][None, :]
    cn_term = jnp.sum(acc, axis=1)
    xij = jnp.sum(xi_ref[...] * xj_ref[...] * w1_ref[...], axis=1)
    out_ref[0, 0, :] = cn_term + xij + b_ref[0, 0]


def kernel(x, adj, tar_ei, boolen, W_xslin, b_xslin):
    del boolen
    x = x.astype(jnp.float32)
    adj0 = adj[0].astype(jnp.int32)
    adj1 = adj[1].astype(jnp.int32)
    t0 = tar_ei[0].astype(jnp.int32)
    t1 = tar_ei[1].astype(jnp.int32)
    w1 = W_xslin[0, :D_FEAT].reshape(1, D_FEAT)
    w2 = W_xslin[0, D_FEAT:].reshape(1, D_FEAT)
    b_arr = b_xslin.reshape(1, 1)

    mesh = plsc.VectorSubcoreMesh(core_axis_name="c", subcore_axis_name="s")

    build = pl.kernel(
        _build_body,
        out_type=jax.ShapeDtypeStruct((N_NODES * WORDS,), jnp.int32),
        mesh=mesh,
        scratch_types=[
            pltpu.VMEM((MEGA,), jnp.int32),
            pltpu.VMEM((MEGA,), jnp.int32),
            pltpu.VMEM((MEGA_ROWS, MROW), jnp.int32),
            pltpu.VMEM((MEGA_ROWS, MROW), jnp.int32),
            pltpu.VMEM((ZCHUNK,), jnp.int32),
            pltpu.VMEM((FCHUNK,), jnp.int32),
            pltpu.VMEM_SHARED((SPMEM_WORDS,), jnp.int32),
            pltpu.SemaphoreType.DMA,
        ],
    )
    c_flat = build(adj0, adj1)
    c_2d = c_flat.reshape(N_NODES, WORDS)

    gather = pl.kernel(
        _gather_body,
        out_type=(
            jax.ShapeDtypeStruct((N_TAR, WORDS), jnp.int32),
            jax.ShapeDtypeStruct((N_TAR, WORDS), jnp.int32),
            jax.ShapeDtypeStruct((N_TAR, D_FEAT), jnp.float32),
            jax.ShapeDtypeStruct((N_TAR, D_FEAT), jnp.float32),
        ),
        mesh=mesh,
        scratch_types=[
            pltpu.VMEM((GCHUNK,), jnp.int32),
            pltpu.VMEM((GCHUNK, WORDS), jnp.int32),
            pltpu.VMEM((GCHUNK, D_FEAT), jnp.float32),
            pltpu.SemaphoreType.DMA,
        ],
    )
    gi, gj, xi, xj = gather(c_2d, x, t0, t1)

    x_pad = jnp.pad(x, ((0, K_PAD - N_NODES), (0, 0)))
    s_blocks = pl.pallas_call(
        _s_body,
        grid=(K_PAD // 2048,),
        in_specs=[
            pl.BlockSpec((2048, D_FEAT), lambda i: (i, 0)),
            pl.BlockSpec((1, D_FEAT), lambda i: (0, 0)),
        ],
        out_specs=pl.BlockSpec((1, 1, 2048), lambda i: (i, 0, 0)),
        out_shape=jax.ShapeDtypeStruct((K_PAD // 2048, 1, 2048), jnp.float32),
    )(x_pad, w2)
    s_r = s_blocks.reshape(K_PAD)[: WORDS * 8].reshape(WORDS, 8).T

    EB = 512
    out_blocks = pl.pallas_call(
        _dense_body,
        grid=(N_TAR // EB,),
        in_specs=[
            pl.BlockSpec((EB, WORDS), lambda i: (i, 0)),
            pl.BlockSpec((EB, WORDS), lambda i: (i, 0)),
            pl.BlockSpec((EB, D_FEAT), lambda i: (i, 0)),
            pl.BlockSpec((EB, D_FEAT), lambda i: (i, 0)),
            pl.BlockSpec((8, WORDS), lambda i: (0, 0)),
            pl.BlockSpec((1, D_FEAT), lambda i: (0, 0)),
            pl.BlockSpec((1, 1), lambda i: (0, 0)),
        ],
        out_specs=pl.BlockSpec((1, 1, EB), lambda i: (i, 0, 0)),
        out_shape=jax.ShapeDtypeStruct((N_TAR // EB, 1, EB), jnp.float32),
    )(gi, gj, xi, xj, s_r, w1, b_arr)
    return out_blocks.reshape(N_TAR, 1)


# speedup vs baseline: 6.9145x; 1.0457x over previous
import functools

import jax
import jax.numpy as jnp
from jax import lax
from jax.experimental import pallas as pl
from jax.experimental.pallas import tpu as pltpu
from jax.experimental.pallas import tpu_sc as plsc

N_NODES = 10000
D_FEAT = 128
N_EDGES = 320000
N_TAR = 4096

NC = 2
NS = 16
LANES = 16

WORDS = 1280
ROWS_PER_PASS = 1250
N_PASS = 4
BLOCK_WORDS = ROWS_PER_PASS * WORDS
SENT_PAD = 128
SPMEM_WORDS = BLOCK_WORDS + SENT_PAD
ZSTRIPE = SPMEM_WORDS // NS
FSTRIPE = BLOCK_WORDS // NS
E_PER_TEC = N_EDGES // NS
MEGA = 2000
N_MEGA = E_PER_TEC // MEGA
MROW = 80
MEGA_ROWS = MEGA // MROW

G_ROWS = 2 * N_TAR
G_PER_TEC = N_TAR // (NC * NS)
GCHUNK = 64
NGCH = G_PER_TEC // GCHUNK

K_PAD = WORDS * 8

ZCHUNK = 4096
FCHUNK = FSTRIPE // 20


def _build_body(src_hbm, dst_hbm, c_hbm, src_v, dst_v, idx_v, val_v, idx2_v,
                val2_v, zbuf, bounce, spmem, sem):
    cid = lax.axis_index("c")
    sid = lax.axis_index("s")

    def zinit(i, _):
        zbuf[pl.ds(i * LANES, LANES)] = jnp.zeros((LANES,), jnp.int32)
        return 0
    lax.fori_loop(0, ZCHUNK // LANES, zinit, 0)

    for p in range(N_PASS):
        zoff = sid * ZSTRIPE
        nzfull = ZSTRIPE // ZCHUNK
        for i in range(nzfull):
            pltpu.sync_copy(zbuf, spmem.at[pl.ds(zoff + i * ZCHUNK, ZCHUNK)])
        pltpu.sync_copy(zbuf.at[pl.ds(0, ZSTRIPE - nzfull * ZCHUNK)],
                        spmem.at[pl.ds(zoff + nzfull * ZCHUNK,
                                       ZSTRIPE - nzfull * ZCHUNK)])
        plsc.subcore_barrier()

        row_lo = cid * (ROWS_PER_PASS * N_PASS) + p * ROWS_PER_PASS
        row_hi = row_lo + ROWS_PER_PASS
        sent = BLOCK_WORDS + sid * 8

        def do_mega(m, idx_b, val_b, drain):
            ebase = sid * E_PER_TEC + m * MEGA
            pltpu.sync_copy(src_hbm.at[pl.ds(ebase, MEGA)], src_v)
            pltpu.sync_copy(dst_hbm.at[pl.ds(ebase, MEGA)], dst_v)

            def row_body(q, _):
                for v in range(MROW // LANES):
                    r = src_v[pl.ds(q * MROW + v * LANES, LANES)]
                    c = dst_v[pl.ds(q * MROW + v * LANES, LANES)]
                    inr = (r >= row_lo) & (r < row_hi)
                    addr = ((r - row_lo) * WORDS
                            + lax.shift_right_logical(c, 3))
                    addr = jnp.where(inr, addr, sent)
                    shift = lax.shift_left(jnp.bitwise_and(c, 7), 2)
                    val = jnp.where(inr, lax.shift_left(1, shift), 0)
                    idx_b[q, pl.ds(v * LANES, LANES)] = addr
                    val_b[q, pl.ds(v * LANES, LANES)] = val
                return 0
            lax.fori_loop(0, MEGA // MROW, row_body, 0)

            if drain is not None:
                didx, dval = drain
                for q in range(MEGA // MROW):
                    pltpu.make_async_copy(
                        dval.at[q], spmem.at[didx.at[q]], sem).wait()
            for q in range(MEGA // MROW):
                pltpu.async_copy(val_b.at[q], spmem.at[idx_b.at[q]], sem,
                                 add=True)

        do_mega(0, idx_v, val_v, None)

        def pair_body(k, _):
            do_mega(2 * k + 1, idx2_v, val2_v, (idx_v, val_v))
            do_mega(2 * k + 2, idx_v, val_v, (idx2_v, val2_v))
            return 0
        lax.fori_loop(0, (N_MEGA - 2) // 2, pair_body, 0)

        do_mega(N_MEGA - 1, idx2_v, val2_v, (idx_v, val_v))
        for q in range(MEGA // MROW):
            pltpu.make_async_copy(
                val2_v.at[q], spmem.at[idx2_v.at[q]], sem).wait()
        plsc.subcore_barrier()

        foff = sid * FSTRIPE
        for f in range(FSTRIPE // FCHUNK):
            pltpu.sync_copy(spmem.at[pl.ds(foff + f * FCHUNK, FCHUNK)],
                            bounce)
            pltpu.sync_copy(
                bounce,
                c_hbm.at[pl.ds(row_lo * WORDS + foff + f * FCHUNK, FCHUNK)])
        plsc.subcore_barrier()


def _gather_body(c_hbm, x_hbm, t0_hbm, t1_hbm, gi_hbm, gj_hbm, xi_hbm, xj_hbm,
                 idx_v, rows_v, xrows_v, sem):
    cid = lax.axis_index("c")
    sid = lax.axis_index("s")
    wid = cid * NS + sid
    base = wid * G_PER_TEC
    for t_hbm, g_hbm, xg_hbm in ((t0_hbm, gi_hbm, xi_hbm),
                                 (t1_hbm, gj_hbm, xj_hbm)):
        for ch in range(NGCH):
            off = base + ch * GCHUNK
            pltpu.sync_copy(t_hbm.at[pl.ds(off, GCHUNK)], idx_v)
            pltpu.async_copy(c_hbm.at[idx_v], rows_v, sem).wait()
            pltpu.sync_copy(rows_v, g_hbm.at[pl.ds(off, GCHUNK)])
            pltpu.async_copy(x_hbm.at[idx_v], xrows_v, sem).wait()
            pltpu.sync_copy(xrows_v, xg_hbm.at[pl.ds(off, GCHUNK)])


def _s_body(x_ref, w2_ref, out_ref):
    out_ref[0, 0, :] = jnp.sum(x_ref[...] * w2_ref[...], axis=1)


def _dense_body(gi_ref, gj_ref, xi_ref, xj_ref, sr_ref, w1_ref, b_ref,
                out_ref):
    gi = gi_ref[...]
    gj = gj_ref[...]
    mask_const = jnp.int32(0x11111111)
    zi = gi | lax.shift_right_logical(gi, 1)
    zi = (zi | lax.shift_right_logical(zi, 2)) & mask_const
    zj = gj | lax.shift_right_logical(gj, 1)
    zj = (zj | lax.shift_right_logical(zj, 2)) & mask_const
    m = zi & zj
    acc = jnp.zeros(gi.shape, jnp.float32)
    for t in range(8):
        bit = lax.shift_right_logical(m, 4 * t) & 1
        acc = acc + bit.astype(jnp.float32) * sr_ref[t, :][None, :]
    cn_term = jnp.sum(acc, axis=1)
    xij = jnp.sum(xi_ref[...] * xj_ref[...] * w1_ref[...], axis=1)
    out_ref[0, 0, :] = cn_term + xij + b_ref[0, 0]


def kernel(x, adj, tar_ei, boolen, W_xslin, b_xslin):
    del boolen
    x = x.astype(jnp.float32)
    adj0 = adj[0].astype(jnp.int32)
    adj1 = adj[1].astype(jnp.int32)
    t0 = tar_ei[0].astype(jnp.int32)
    t1 = tar_ei[1].astype(jnp.int32)
    w1 = W_xslin[0, :D_FEAT].reshape(1, D_FEAT)
    w2 = W_xslin[0, D_FEAT:].reshape(1, D_FEAT)
    b_arr = b_xslin.reshape(1, 1)

    mesh = plsc.VectorSubcoreMesh(core_axis_name="c", subcore_axis_name="s")

    build = pl.kernel(
        _build_body,
        out_type=jax.ShapeDtypeStruct((N_NODES * WORDS,), jnp.int32),
        mesh=mesh,
        scratch_types=[
            pltpu.VMEM((MEGA,), jnp.int32),
            pltpu.VMEM((MEGA,), jnp.int32),
            pltpu.VMEM((MEGA_ROWS, MROW), jnp.int32),
            pltpu.VMEM((MEGA_ROWS, MROW), jnp.int32),
            pltpu.VMEM((MEGA_ROWS, MROW), jnp.int32),
            pltpu.VMEM((MEGA_ROWS, MROW), jnp.int32),
            pltpu.VMEM((ZCHUNK,), jnp.int32),
            pltpu.VMEM((FCHUNK,), jnp.int32),
            pltpu.VMEM_SHARED((SPMEM_WORDS,), jnp.int32),
            pltpu.SemaphoreType.DMA,
        ],
    )
    c_flat = build(adj0, adj1)
    c_2d = c_flat.reshape(N_NODES, WORDS)

    gather = pl.kernel(
        _gather_body,
        out_type=(
            jax.ShapeDtypeStruct((N_TAR, WORDS), jnp.int32),
            jax.ShapeDtypeStruct((N_TAR, WORDS), jnp.int32),
            jax.ShapeDtypeStruct((N_TAR, D_FEAT), jnp.float32),
            jax.ShapeDtypeStruct((N_TAR, D_FEAT), jnp.float32),
        ),
        mesh=mesh,
        scratch_types=[
            pltpu.VMEM((GCHUNK,), jnp.int32),
            pltpu.VMEM((GCHUNK, WORDS), jnp.int32),
            pltpu.VMEM((GCHUNK, D_FEAT), jnp.float32),
            pltpu.SemaphoreType.DMA,
        ],
    )
    gi, gj, xi, xj = gather(c_2d, x, t0, t1)

    s_blocks = pl.pallas_call(
        _s_body,
        grid=(N_NODES // 2000,),
        in_specs=[
            pl.BlockSpec((2000, D_FEAT), lambda i: (i, 0)),
            pl.BlockSpec((1, D_FEAT), lambda i: (0, 0)),
        ],
        out_specs=pl.BlockSpec((1, 1, 2000), lambda i: (i, 0, 0)),
        out_shape=jax.ShapeDtypeStruct((N_NODES // 2000, 1, 2000),
                                       jnp.float32),
    )(x, w2)
    s_pad = jnp.pad(s_blocks.reshape(N_NODES), (0, K_PAD - N_NODES))
    s_r = s_pad.reshape(WORDS, 8).T

    EB = 512
    out_blocks = pl.pallas_call(
        _dense_body,
        grid=(N_TAR // EB,),
        in_specs=[
            pl.BlockSpec((EB, WORDS), lambda i: (i, 0)),
            pl.BlockSpec((EB, WORDS), lambda i: (i, 0)),
            pl.BlockSpec((EB, D_FEAT), lambda i: (i, 0)),
            pl.BlockSpec((EB, D_FEAT), lambda i: (i, 0)),
            pl.BlockSpec((8, WORDS), lambda i: (0, 0)),
            pl.BlockSpec((1, D_FEAT), lambda i: (0, 0)),
            pl.BlockSpec((1, 1), lambda i: (0, 0)),
        ],
        out_specs=pl.BlockSpec((1, 1, EB), lambda i: (i, 0, 0)),
        out_shape=jax.ShapeDtypeStruct((N_TAR // EB, 1, EB), jnp.float32),
    )(gi, gj, xi, xj, s_r, w1, b_arr)
    return out_blocks.reshape(N_TAR, 1)
